# baseline (speedup 1.0000x reference)
import jax
import jax.numpy as jnp
from jax import lax
from jax.experimental import pallas as pl
from jax.experimental.pallas import tpu as pltpu
from jax.experimental.pallas import tpu_sc as plsc

B = 64
V = 100000
K = 50
L = 16
NTILES = 32
ROWS_PER_TILE = B // NTILES
UNROLL = 10
SLOTCAP = 1024

_I32_MIN = -(2 ** 31)
_I32_MAX = 2 ** 31 - 1


def _ukey(x):
    b = lax.bitcast_convert_type(x, jnp.int32)
    return b ^ ((b >> 31) | jnp.int32(_I32_MIN))


def _scan_hist(hist_ref, nbuckets, a0, k, iota, j0=None):
    nchunks = nbuckets // L

    def cond(st):
        j, _, found, _, _ = st
        return jnp.logical_and(j < nchunks, jnp.logical_not(found))

    def body(st):
        j, a, found, bsel, asel = st
        jj = nchunks - 1 - j
        c = hist_ref[pl.ds(jj * L, L)]
        rc = lax.rev(c, (0,))
        cs = jnp.cumsum(rc)
        svec = a + cs
        total = svec[L - 1]
        crossed = total >= k
        f = plsc.all_reduce_ffs(svec >= k)[0]
        above = a + jnp.sum(jnp.where(iota < f, rc, 0))
        bnew = jj * L + (L - 1) - f
        bsel = jnp.where(crossed, bnew, bsel)
        asel = jnp.where(crossed, above, asel)
        return (j + 1, total, crossed, bsel, asel)

    jstart = jnp.int32(0) if j0 is None else j0
    st = lax.while_loop(
        cond, body,
        (jstart, a0, jnp.bool_(False), jnp.int32(0), jnp.int32(0)))
    return st[3], st[4]


def _body(logits_hbm, conf_hbm, idx_hbm,
          row_v, cand, cnt1, cnt2, cnt3, esum3, stage_f, stage_i, sem):
    c = lax.axis_index("c")
    s = lax.axis_index("s")
    wid = s * 2 + c
    iota = lax.iota(jnp.int32, L)
    ones_i = jnp.ones((L,), jnp.int32)
    zeros_i = jnp.zeros((L,), jnp.int32)
    zeros_f = jnp.zeros((L,), jnp.float32)
    kk = jnp.int32(K)

    for r in range(ROWS_PER_TILE):
        row = wid + r * NTILES
        cp = pltpu.async_copy(logits_hbm.at[row], row_v, sem)

        @plsc.parallel_loop(0, 4096, step=L)
        def _zero12(i):
            cnt1[pl.ds(i, L)] = zeros_i
            cnt2[pl.ds(i, L)] = zeros_i

        @plsc.parallel_loop(0, 256, step=L)
        def _zero3(i):
            cnt3[pl.ds(i, L)] = zeros_i
            esum3[pl.ds(i, L)] = zeros_f

        cp.wait()

        carry0 = (jnp.full((L,), -jnp.inf, jnp.float32),
                  jnp.zeros((L,), jnp.int32))

        @plsc.parallel_loop(0, V, step=L, unroll=UNROLL, carry=carry0)
        def p1(i, cr):
            lmax, lidx = cr
            x = row_v[pl.ds(i, L)]
            uk = _ukey(x)
            b1 = lax.shift_right_logical(uk, 20)
            plsc.addupdate_scatter(cnt1, [b1], ones_i)
            pos = i + iota
            upd = x > lmax
            tie = x == lmax
            lidx = jnp.where(
                upd, pos, jnp.where(tie, jnp.minimum(lidx, pos), lidx))
            lmax = jnp.maximum(lmax, x)
            return (lmax, lidx)

        lmax, lidx = p1
        m = jnp.max(lmax)
        amax = jnp.min(jnp.where(lmax == m, lidx, jnp.int32(_I32_MAX)))

        mb = lax.shift_right_logical(_ukey(jnp.full((L, ), m))[0], 20)
        j0 = (4095 - mb) >> 4
        b1sel, a1 = _scan_hist(cnt1, 4096, jnp.int32(0), kk, iota, j0=j0)

        @plsc.parallel_loop(0, V, step=L, unroll=UNROLL, carry=zeros_i)
        def p2(i, slotv16):
            x = row_v[pl.ds(i, L)]
            uk = _ukey(x)
            hit = lax.shift_right_logical(uk, 20) >= b1sel
            anyv = plsc.all_reduce_population_count(hit) > 0
            addr = jnp.where(anyv,
                             jnp.minimum(slotv16, (SLOTCAP - 1) * L),
                             SLOTCAP * L)
            plsc.store_scatter(cand, [addr | iota], x)
            return slotv16 + jnp.where(anyv, L, 0)

        slot_end = lax.shift_right_logical(p2[0], 4)
        nslots = jnp.minimum(slot_end, jnp.int32(SLOTCAP))
        ovf = slot_end > jnp.int32(SLOTCAP)

        def cand2(_):
            def bdy(sl, _2):
                x = cand[pl.ds(sl * L, L)]
                uk = _ukey(x)
                inb = lax.shift_right_logical(uk, 20) == b1sel
                sub = lax.shift_right_logical(uk, 8) & 0xFFF
                plsc.addupdate_scatter(cnt2, [sub], ones_i, mask=inb)
                return 0
            lax.fori_loop(0, nslots, bdy, 0)
            return 0

        def full2(_):
            @plsc.parallel_loop(0, V, step=L, unroll=UNROLL)
            def _f2(i):
                x = row_v[pl.ds(i, L)]
                uk = _ukey(x)
                inb = lax.shift_right_logical(uk, 20) == b1sel
                sub = lax.shift_right_logical(uk, 8) & 0xFFF
                plsc.addupdate_scatter(cnt2, [sub], ones_i, mask=inb)
            return 0

        lax.cond(ovf, full2, cand2, 0)

        b2sel, a2 = _scan_hist(cnt2, 4096, a1, kk, iota)
        p2pref = (b1sel << 12) | b2sel

        def cand3(_):
            def bdy(sl, acc):
                x = cand[pl.ds(sl * L, L)]
                uk = _ukey(x)
                top24 = lax.shift_right_logical(uk, 8)
                e = jnp.exp(x - m)
                eq = top24 == p2pref
                acc = acc + jnp.where(top24 > p2pref, e, 0.0)
                low = uk & 0xFF
                plsc.addupdate_scatter(cnt3, [low], ones_i, mask=eq)
                plsc.addupdate_scatter(esum3, [low], e, mask=eq)
                return acc
            return lax.fori_loop(0, nslots, bdy, zeros_f)

        def full3(_):
            @plsc.parallel_loop(0, V, step=L, unroll=UNROLL, carry=zeros_f)
            def p3(i, acc):
                x = row_v[pl.ds(i, L)]
                uk = _ukey(x)
                top24 = lax.shift_right_logical(uk, 8)
                e = jnp.exp(x - m)
                eq = top24 == p2pref
                acc = acc + jnp.where(top24 > p2pref, e, 0.0)
                low = uk & 0xFF
                plsc.addupdate_scatter(cnt3, [low], ones_i, mask=eq)
                plsc.addupdate_scatter(esum3, [low], e, mask=eq)
                return acc
            return p3

        s_hi = jnp.sum(lax.cond(ovf, full3, cand3, 0))
        b3sel, _ = _scan_hist(cnt3, 256, a2, kk, iota)

        def tail(j, acc2):
            ev = esum3[pl.ds(j * L, L)]
            keep = (j * L + iota) >= b3sel
            return acc2 + jnp.where(keep, ev, 0.0)

        s_tail = jnp.sum(lax.fori_loop(0, 256 // L, tail, zeros_f))

        stage_f[...] = 1.0 / jnp.full((L,), s_hi + s_tail)
        stage_i[...] = jnp.full((L,), amax)
        pltpu.sync_copy(stage_f, conf_hbm.at[row])
        pltpu.sync_copy(stage_i, idx_hbm.at[row])


@jax.jit
def _run(logits):
    mesh = plsc.VectorSubcoreMesh(core_axis_name="c", subcore_axis_name="s")
    fn = pl.kernel(
        _body,
        out_type=(jax.ShapeDtypeStruct((B, L), jnp.float32),
                  jax.ShapeDtypeStruct((B, L), jnp.int32)),
        mesh=mesh,
        scratch_types=(
            pltpu.VMEM((V,), jnp.float32),
            pltpu.VMEM(((SLOTCAP + 1) * L,), jnp.float32),
            pltpu.VMEM((4096,), jnp.int32),
            pltpu.VMEM((4096,), jnp.int32),
            pltpu.VMEM((256,), jnp.int32),
            pltpu.VMEM((256,), jnp.float32),
            pltpu.VMEM((L,), jnp.float32),
            pltpu.VMEM((L,), jnp.int32),
            pltpu.SemaphoreType.DMA,
        ),
        compiler_params=pltpu.CompilerParams(needs_layout_passes=False),
    )
    return fn(logits)


def kernel(logits, top_k):
    del top_k
    conf, idx = _run(logits)
    c0 = conf[:, 0]
    return (c0, idx[:, 0], c0)


# speedup vs baseline: 1.0452x; 1.0452x over previous
import jax
import jax.numpy as jnp
from jax import lax
from jax.experimental import pallas as pl
from jax.experimental.pallas import tpu as pltpu
from jax.experimental.pallas import tpu_sc as plsc

B = 64
V = 100000
K = 50
L = 16
NTILES = 32
ROWS_PER_TILE = B // NTILES
UNROLL = 5
SLOTCAP = 1024

_I32_MIN = -(2 ** 31)
_I32_MAX = 2 ** 31 - 1


def _ukey(x):
    b = lax.bitcast_convert_type(x, jnp.int32)
    return b ^ ((b >> 31) | jnp.int32(_I32_MIN))


def _scan_hist(hist_ref, nbuckets, a0, k, iota, j0=None):
    nchunks = nbuckets // L

    def cond(st):
        j, _, found, _, _ = st
        return jnp.logical_and(j < nchunks, jnp.logical_not(found))

    def body(st):
        j, a, found, bsel, asel = st
        jj = nchunks - 1 - j
        c = hist_ref[pl.ds(jj * L, L)]
        rc = lax.rev(c, (0,))
        cs = jnp.cumsum(rc)
        svec = a + cs
        total = svec[L - 1]
        crossed = total >= k
        f = plsc.all_reduce_ffs(svec >= k)[0]
        above = a + jnp.sum(jnp.where(iota < f, rc, 0))
        bnew = jj * L + (L - 1) - f
        bsel = jnp.where(crossed, bnew, bsel)
        asel = jnp.where(crossed, above, asel)
        return (j + 1, total, crossed, bsel, asel)

    jstart = jnp.int32(0) if j0 is None else j0
    st = lax.while_loop(
        cond, body,
        (jstart, a0, jnp.bool_(False), jnp.int32(0), jnp.int32(0)))
    return st[3], st[4]


def _body(logits_hbm, conf_hbm, idx_hbm,
          row_v, cand, cnt1, cnt2, cnt3, esum3, stage_f, stage_i, sem):
    c = lax.axis_index("c")
    s = lax.axis_index("s")
    wid = s * 2 + c
    iota = lax.iota(jnp.int32, L)
    ones_i = jnp.ones((L,), jnp.int32)
    zeros_i = jnp.zeros((L,), jnp.int32)
    zeros_f = jnp.zeros((L,), jnp.float32)
    kk = jnp.int32(K)

    pltpu.async_copy(logits_hbm.at[wid], row_v, sem)

    for r in range(ROWS_PER_TILE):
        row = wid + r * NTILES

        @plsc.parallel_loop(0, 4096, step=L)
        def _zero12(i):
            cnt1[pl.ds(i, L)] = zeros_i
            cnt2[pl.ds(i, L)] = zeros_i

        @plsc.parallel_loop(0, 256, step=L)
        def _zero3(i):
            cnt3[pl.ds(i, L)] = zeros_i
            esum3[pl.ds(i, L)] = zeros_f

        pltpu.make_async_copy(logits_hbm.at[row], row_v, sem).wait()

        carry0 = (jnp.full((L,), -jnp.inf, jnp.float32),
                  jnp.zeros((L,), jnp.int32))

        @plsc.parallel_loop(0, V, step=L, unroll=UNROLL, carry=carry0)
        def p1(i, cr):
            lmax, lidx = cr
            x = row_v[pl.ds(i, L)]
            uk = _ukey(x)
            b1 = lax.shift_right_logical(uk, 20)
            plsc.addupdate_scatter(cnt1, [b1], ones_i)
            pos = i + iota
            upd = x > lmax
            tie = x == lmax
            lidx = jnp.where(
                upd, pos, jnp.where(tie, jnp.minimum(lidx, pos), lidx))
            lmax = jnp.maximum(lmax, x)
            return (lmax, lidx)

        lmax, lidx = p1
        m = jnp.max(lmax)
        amax = jnp.min(jnp.where(lmax == m, lidx, jnp.int32(_I32_MAX)))

        mb = lax.shift_right_logical(_ukey(jnp.full((L, ), m))[0], 20)
        j0 = (4095 - mb) >> 4
        b1sel, a1 = _scan_hist(cnt1, 4096, jnp.int32(0), kk, iota, j0=j0)

        @plsc.parallel_loop(0, V, step=L, unroll=UNROLL, carry=zeros_i)
        def p2(i, slotv16):
            x = row_v[pl.ds(i, L)]
            uk = _ukey(x)
            hit = lax.shift_right_logical(uk, 20) >= b1sel
            anyv = plsc.all_reduce_population_count(hit) > 0
            addr = jnp.where(anyv,
                             jnp.minimum(slotv16, (SLOTCAP - 1) * L),
                             SLOTCAP * L)
            plsc.store_scatter(cand, [addr | iota], x)
            return slotv16 + jnp.where(anyv, L, 0)

        slot_end = lax.shift_right_logical(p2[0], 4)
        nslots = jnp.minimum(slot_end, jnp.int32(SLOTCAP))
        ovf = slot_end > jnp.int32(SLOTCAP)

        def cand2(_):
            def bdy(sl, _2):
                x = cand[pl.ds(sl * L, L)]
                uk = _ukey(x)
                inb = lax.shift_right_logical(uk, 20) == b1sel
                sub = lax.shift_right_logical(uk, 8) & 0xFFF
                plsc.addupdate_scatter(cnt2, [sub], ones_i, mask=inb)
                return 0
            lax.fori_loop(0, nslots, bdy, 0)
            return 0

        def full2(_):
            @plsc.parallel_loop(0, V, step=L, unroll=UNROLL)
            def _f2(i):
                x = row_v[pl.ds(i, L)]
                uk = _ukey(x)
                inb = lax.shift_right_logical(uk, 20) == b1sel
                sub = lax.shift_right_logical(uk, 8) & 0xFFF
                plsc.addupdate_scatter(cnt2, [sub], ones_i, mask=inb)
            return 0

        lax.cond(ovf, full2, cand2, 0)

        b2sel, a2 = _scan_hist(cnt2, 4096, a1, kk, iota)
        p2pref = (b1sel << 12) | b2sel

        nxt = wid + (r + 1) * NTILES

        def cand3(_):
            if r + 1 < ROWS_PER_TILE:
                pltpu.async_copy(logits_hbm.at[nxt], row_v, sem)

            def bdy(sl, acc):
                x = cand[pl.ds(sl * L, L)]
                uk = _ukey(x)
                top24 = lax.shift_right_logical(uk, 8)
                e = jnp.exp(x - m)
                eq = top24 == p2pref
                acc = acc + jnp.where(top24 > p2pref, e, 0.0)
                low = uk & 0xFF
                plsc.addupdate_scatter(cnt3, [low], ones_i, mask=eq)
                plsc.addupdate_scatter(esum3, [low], e, mask=eq)
                return acc
            return lax.fori_loop(0, nslots, bdy, zeros_f)

        def full3(_):
            @plsc.parallel_loop(0, V, step=L, unroll=UNROLL, carry=zeros_f)
            def p3(i, acc):
                x = row_v[pl.ds(i, L)]
                uk = _ukey(x)
                top24 = lax.shift_right_logical(uk, 8)
                e = jnp.exp(x - m)
                eq = top24 == p2pref
                acc = acc + jnp.where(top24 > p2pref, e, 0.0)
                low = uk & 0xFF
                plsc.addupdate_scatter(cnt3, [low], ones_i, mask=eq)
                plsc.addupdate_scatter(esum3, [low], e, mask=eq)
                return acc

            if r + 1 < ROWS_PER_TILE:
                pltpu.async_copy(logits_hbm.at[nxt], row_v, sem)
            return p3

        s_hi = jnp.sum(lax.cond(ovf, full3, cand3, 0))
        b3sel, _ = _scan_hist(cnt3, 256, a2, kk, iota)

        def tail(j, acc2):
            ev = esum3[pl.ds(j * L, L)]
            keep = (j * L + iota) >= b3sel
            return acc2 + jnp.where(keep, ev, 0.0)

        s_tail = jnp.sum(lax.fori_loop(0, 256 // L, tail, zeros_f))

        stage_f[...] = 1.0 / jnp.full((L,), s_hi + s_tail)
        stage_i[...] = jnp.full((L,), amax)
        pltpu.sync_copy(stage_f, conf_hbm.at[row])
        pltpu.sync_copy(stage_i, idx_hbm.at[row])


@jax.jit
def _run(logits):
    mesh = plsc.VectorSubcoreMesh(core_axis_name="c", subcore_axis_name="s")
    fn = pl.kernel(
        _body,
        out_type=(jax.ShapeDtypeStruct((B, L), jnp.float32),
                  jax.ShapeDtypeStruct((B, L), jnp.int32)),
        mesh=mesh,
        scratch_types=(
            pltpu.VMEM((V,), jnp.float32),
            pltpu.VMEM(((SLOTCAP + 1) * L,), jnp.float32),
            pltpu.VMEM((4096,), jnp.int32),
            pltpu.VMEM((4096,), jnp.int32),
            pltpu.VMEM((256,), jnp.int32),
            pltpu.VMEM((256,), jnp.float32),
            pltpu.VMEM((L,), jnp.float32),
            pltpu.VMEM((L,), jnp.int32),
            pltpu.SemaphoreType.DMA,
        ),
        compiler_params=pltpu.CompilerParams(needs_layout_passes=False),
    )
    return fn(logits)


def kernel(logits, top_k):
    del top_k
    conf, idx = _run(logits)
    c0 = conf[:, 0]
    return (c0, idx[:, 0], c0)


# speedup vs baseline: 1.0607x; 1.0149x over previous
import jax
import jax.numpy as jnp
from jax import lax
from jax.experimental import pallas as pl
from jax.experimental.pallas import tpu as pltpu
from jax.experimental.pallas import tpu_sc as plsc

B = 64
V = 100000
K = 50
L = 16
NTILES = 32
ROWS_PER_TILE = B // NTILES
UNROLL = 5
SLOTCAP = 1024

_I32_MIN = -(2 ** 31)
_I32_MAX = 2 ** 31 - 1


def _ukey(x):
    b = lax.bitcast_convert_type(x, jnp.int32)
    return b ^ ((b >> 31) | jnp.int32(_I32_MIN))


def _scan_hist(hist_ref, nbuckets, a0, k, iota, j0=None):
    nchunks = nbuckets // L

    def cond(st):
        j, _, found, _, _ = st
        return jnp.logical_and(j < nchunks, jnp.logical_not(found))

    def body(st):
        j, a, found, bsel, asel = st
        jj = nchunks - 1 - j
        c = hist_ref[pl.ds(jj * L, L)]
        rc = lax.rev(c, (0,))
        cs = jnp.cumsum(rc)
        svec = a + cs
        total = svec[L - 1]
        crossed = total >= k
        f = plsc.all_reduce_ffs(svec >= k)[0]
        above = a + jnp.sum(jnp.where(iota < f, rc, 0))
        bnew = jj * L + (L - 1) - f
        bsel = jnp.where(crossed, bnew, bsel)
        asel = jnp.where(crossed, above, asel)
        return (j + 1, total, crossed, bsel, asel)

    jstart = jnp.int32(0) if j0 is None else j0
    st = lax.while_loop(
        cond, body,
        (jstart, a0, jnp.bool_(False), jnp.int32(0), jnp.int32(0)))
    return st[3], st[4]


def _body(logits_hbm, conf_hbm, idx_hbm,
          row_v, cand, cnt1, cnt2, cnt3, esum3, stage_f, stage_i, sem):
    c = lax.axis_index("c")
    s = lax.axis_index("s")
    wid = s * 2 + c
    iota = lax.iota(jnp.int32, L)
    ones_i = jnp.ones((L,), jnp.int32)
    zeros_i = jnp.zeros((L,), jnp.int32)
    zeros_f = jnp.zeros((L,), jnp.float32)
    kk = jnp.int32(K)

    pltpu.async_copy(logits_hbm.at[wid], row_v, sem)

    for r in range(ROWS_PER_TILE):
        row = wid + r * NTILES

        @plsc.parallel_loop(0, 4096, step=L)
        def _zero12(i):
            cnt1[pl.ds(i, L)] = zeros_i
            cnt2[pl.ds(i, L)] = zeros_i

        @plsc.parallel_loop(0, 256, step=L)
        def _zero3(i):
            cnt3[pl.ds(i, L)] = zeros_i
            esum3[pl.ds(i, L)] = zeros_f

        pltpu.make_async_copy(logits_hbm.at[row], row_v, sem).wait()

        @plsc.parallel_loop(0, V, step=L, unroll=UNROLL,
                            carry=jnp.full((L,), -jnp.inf, jnp.float32))
        def p1(i, lmax):
            x = row_v[pl.ds(i, L)]
            uk = _ukey(x)
            b1 = lax.shift_right_logical(uk, 20)
            plsc.addupdate_scatter(cnt1, [b1], ones_i)
            return jnp.maximum(lmax, x)

        m = jnp.max(p1)
        m_vec = jnp.full((L,), m)

        mb = lax.shift_right_logical(_ukey(jnp.full((L, ), m))[0], 20)
        j0 = (4095 - mb) >> 4
        b1sel, a1 = _scan_hist(cnt1, 4096, jnp.int32(0), kk, iota, j0=j0)

        carry2 = (zeros_i, jnp.full((L,), _I32_MAX, jnp.int32))

        @plsc.parallel_loop(0, V, step=L, unroll=UNROLL, carry=carry2)
        def p2(i, cr):
            slotv16, mp = cr
            x = row_v[pl.ds(i, L)]
            uk = _ukey(x)
            hit = lax.shift_right_logical(uk, 20) >= b1sel
            anyv = plsc.all_reduce_population_count(hit) > 0
            addr = jnp.where(anyv,
                             jnp.minimum(slotv16, (SLOTCAP - 1) * L),
                             SLOTCAP * L)
            plsc.store_scatter(cand, [addr | iota], x)
            mp = jnp.minimum(mp, jnp.where(x == m_vec, i + iota,
                                           jnp.int32(_I32_MAX)))
            return (slotv16 + jnp.where(anyv, L, 0), mp)

        slotv16f, mpf = p2
        amax = jnp.min(mpf)
        slot_end = lax.shift_right_logical(slotv16f[0], 4)
        nslots = jnp.minimum(slot_end, jnp.int32(SLOTCAP))
        ovf = slot_end > jnp.int32(SLOTCAP)

        def cand2(_):
            def bdy(sl, _2):
                x = cand[pl.ds(sl * L, L)]
                uk = _ukey(x)
                inb = lax.shift_right_logical(uk, 20) == b1sel
                sub = lax.shift_right_logical(uk, 8) & 0xFFF
                plsc.addupdate_scatter(cnt2, [sub], ones_i, mask=inb)
                return 0
            lax.fori_loop(0, nslots, bdy, 0)
            return 0

        def full2(_):
            @plsc.parallel_loop(0, V, step=L, unroll=UNROLL)
            def _f2(i):
                x = row_v[pl.ds(i, L)]
                uk = _ukey(x)
                inb = lax.shift_right_logical(uk, 20) == b1sel
                sub = lax.shift_right_logical(uk, 8) & 0xFFF
                plsc.addupdate_scatter(cnt2, [sub], ones_i, mask=inb)
            return 0

        lax.cond(ovf, full2, cand2, 0)

        b2sel, a2 = _scan_hist(cnt2, 4096, a1, kk, iota)
        p2pref = (b1sel << 12) | b2sel

        nxt = wid + (r + 1) * NTILES

        def cand3(_):
            if r + 1 < ROWS_PER_TILE:
                pltpu.async_copy(logits_hbm.at[nxt], row_v, sem)

            def bdy(sl, acc):
                x = cand[pl.ds(sl * L, L)]
                uk = _ukey(x)
                top24 = lax.shift_right_logical(uk, 8)
                e = jnp.exp(x - m)
                eq = top24 == p2pref
                acc = acc + jnp.where(top24 > p2pref, e, 0.0)
                low = uk & 0xFF
                plsc.addupdate_scatter(cnt3, [low], ones_i, mask=eq)
                plsc.addupdate_scatter(esum3, [low], e, mask=eq)
                return acc
            return lax.fori_loop(0, nslots, bdy, zeros_f)

        def full3(_):
            @plsc.parallel_loop(0, V, step=L, unroll=UNROLL, carry=zeros_f)
            def p3(i, acc):
                x = row_v[pl.ds(i, L)]
                uk = _ukey(x)
                top24 = lax.shift_right_logical(uk, 8)
                e = jnp.exp(x - m)
                eq = top24 == p2pref
                acc = acc + jnp.where(top24 > p2pref, e, 0.0)
                low = uk & 0xFF
                plsc.addupdate_scatter(cnt3, [low], ones_i, mask=eq)
                plsc.addupdate_scatter(esum3, [low], e, mask=eq)
                return acc

            if r + 1 < ROWS_PER_TILE:
                pltpu.async_copy(logits_hbm.at[nxt], row_v, sem)
            return p3

        s_hi = jnp.sum(lax.cond(ovf, full3, cand3, 0))
        b3sel, _ = _scan_hist(cnt3, 256, a2, kk, iota)

        def tail(j, acc2):
            ev = esum3[pl.ds(j * L, L)]
            keep = (j * L + iota) >= b3sel
            return acc2 + jnp.where(keep, ev, 0.0)

        s_tail = jnp.sum(lax.fori_loop(0, 256 // L, tail, zeros_f))

        stage_f[...] = 1.0 / jnp.full((L,), s_hi + s_tail)
        stage_i[...] = jnp.full((L,), amax)
        pltpu.sync_copy(stage_f, conf_hbm.at[row])
        pltpu.sync_copy(stage_i, idx_hbm.at[row])


@jax.jit
def _run(logits):
    mesh = plsc.VectorSubcoreMesh(core_axis_name="c", subcore_axis_name="s")
    fn = pl.kernel(
        _body,
        out_type=(jax.ShapeDtypeStruct((B, L), jnp.float32),
                  jax.ShapeDtypeStruct((B, L), jnp.int32)),
        mesh=mesh,
        scratch_types=(
            pltpu.VMEM((V,), jnp.float32),
            pltpu.VMEM(((SLOTCAP + 1) * L,), jnp.float32),
            pltpu.VMEM((4096,), jnp.int32),
            pltpu.VMEM((4096,), jnp.int32),
            pltpu.VMEM((256,), jnp.int32),
            pltpu.VMEM((256,), jnp.float32),
            pltpu.VMEM((L,), jnp.float32),
            pltpu.VMEM((L,), jnp.int32),
            pltpu.SemaphoreType.DMA,
        ),
        compiler_params=pltpu.CompilerParams(needs_layout_passes=False),
    )
    return fn(logits)


def kernel(logits, top_k):
    del top_k
    conf, idx = _run(logits)
    c0 = conf[:, 0]
    return (c0, idx[:, 0], c0)
